# baseline (speedup 1.0000x reference)
import jax
import jax.numpy as jnp
from jax import lax
from jax.experimental import pallas as pl
from jax.experimental.pallas import tpu as pltpu
from jax.experimental.pallas import tpu_sc as plsc

NC, NS = 2, 16
NW = NC * NS
EDIM = 32
CTX = 20
NNEGS = 5
NEG = CTX * NNEGS
COLS = CTX + NEG
BPW = 128
RING = 4
OUT8 = 8


def _sc_body(iwgi2, awgi3, in2, out2, scores,
             igbuf, agbuf, eib, rows, accv,
             sem0, sem1, sem2, sem3):
    wid = lax.axis_index("s") * NC + lax.axis_index("c")
    base = wid * BPW
    sems = [sem0, sem1, sem2, sem3]

    pltpu.sync_copy(iwgi2.at[wid], igbuf)
    pltpu.sync_copy(awgi3.at[wid], agbuf)

    pltpu.async_copy(in2.at[igbuf], eib, sem0)
    pltpu.make_async_copy(in2.at[pl.ds(0, BPW)], eib, sem0).wait()

    def fire(b, slot):
        pltpu.async_copy(out2.at[agbuf.at[b]], rows.at[slot], sems[slot])

    def drain(slot):
        pltpu.make_async_copy(out2.at[pl.ds(0, COLS)], rows.at[slot],
                              sems[slot]).wait()

    lane = lax.iota(jnp.int32, 16)
    masks = [lane == j for j in range(16)]
    zv = jnp.zeros((16,), jnp.float32)

    def compute(b, slot, bi):
        e0 = eib[b, pl.ds(0, 16)]
        e1 = eib[b, pl.ds(16, 16)]

        def col_score(c):
            p = rows[slot, c, pl.ds(0, 16)] * e0 \
                + rows[slot, c, pl.ds(16, 16)] * e1
            return jnp.sum(p)

        def per_chunk(ch, c2):
            vec = zv
            for j in range(16):
                s = col_score(ch * 16 + j)
                vec = jnp.where(masks[j], jnp.full((16,), s), vec)
            accv[bi, pl.ds(ch * 16, 16)] = vec
            return c2
        lax.fori_loop(0, COLS // 16, per_chunk, 0)

        vec = zv
        for j in range(8):
            s = col_score(112 + j)
            vec = jnp.where(masks[j], jnp.full((16,), s), vec)
        accv[bi, pl.ds(112, 16)] = vec

    for b in range(RING - 1):
        fire(b, b)

    def octet(ob, carry):
        b0 = ob * OUT8
        for bi in range(8):
            b = b0 + bi
            slot = bi % RING
            drain(slot)
            compute(b, slot, bi)

            @pl.when(b + RING - 1 < BPW)
            def _():
                fire(b + RING - 1, (bi + RING - 1) % RING)
        pltpu.sync_copy(accv, scores.at[pl.ds(base + b0, OUT8)])
        return carry

    lax.fori_loop(0, BPW // OUT8, octet, 0)


def _sc_scores(iword, allwords, in_embed, out_embed):
    B = iword.shape[0]
    mesh = plsc.VectorSubcoreMesh(core_axis_name="c", subcore_axis_name="s",
                                  num_cores=NC, num_subcores=NS)
    f = pl.kernel(
        _sc_body,
        out_type=jax.ShapeDtypeStruct((B, 128), jnp.float32),
        mesh=mesh,
        compiler_params=pltpu.CompilerParams(
            needs_layout_passes=False, use_tc_tiling_on_sc=False),
        scratch_types=[
            pltpu.VMEM((BPW,), jnp.int32),
            pltpu.VMEM((BPW, COLS), jnp.int32),
            pltpu.VMEM((BPW, EDIM), jnp.float32),
            pltpu.VMEM((RING, COLS, EDIM), jnp.float32),
            pltpu.VMEM((OUT8, 128), jnp.float32),
            pltpu.SemaphoreType.DMA,
            pltpu.SemaphoreType.DMA,
            pltpu.SemaphoreType.DMA,
            pltpu.SemaphoreType.DMA,
        ],
    )
    return f(iword.reshape(NW, BPW),
             allwords.reshape(NW, BPW, COLS),
             in_embed,
             out_embed)


def _loss_body(sc_ref, out_ref):
    x = sc_ref[...]
    col = lax.broadcasted_iota(jnp.int32, x.shape, 1)
    xs = jnp.where(col < CTX, x, -x)
    contrib = jnp.where(col < COLS, jnp.log(jax.nn.sigmoid(xs)), 0.0)
    out_ref[...] = jnp.reshape(-jnp.sum(contrib), (1, 1))


def kernel(iword, owords, nwords, in_embed, out_embed):
    B = iword.shape[0]
    iw = iword.astype(jnp.int32)
    aw = jnp.concatenate([owords.astype(jnp.int32),
                          nwords.astype(jnp.int32)], axis=1)
    scores = _sc_scores(iw, aw, in_embed, out_embed)
    tot = pl.pallas_call(
        _loss_body,
        out_shape=jax.ShapeDtypeStruct((1, 1), jnp.float32),
    )(scores)
    return jnp.reshape(tot, ()) / (CTX * B)
